# baseline (speedup 1.0000x reference)
import functools

import jax
import jax.numpy as jnp
from jax import lax
from jax.experimental import pallas as pl
from jax.experimental.pallas import tpu as pltpu
from jax.experimental.pallas import tpu_sc as plsc

N = 10000
D = 128
E = 320000
NACC = 10112
NSC = 2
NSUB = 16
NW = NSC * NSUB
CE = 64
CHN = 160
NBUF = 5
STEPS = CHN // NBUF
EPW = CHN * CE
EPAD = NW * EPW - E
RPS = NACC // NSUB
R = 1000

_mesh = plsc.VectorSubcoreMesh(core_axis_name="c", subcore_axis_name="s")


def _prop_body(g_hbm, idx_hbm, zer_hbm, out_hbm,
               idx_v, rows_v, acc_sh, isem, gsem, ssem):
    cid = lax.axis_index("c")
    sid = lax.axis_index("s")
    r0 = sid * RPS
    pltpu.sync_copy(zer_hbm.at[pl.ds(r0, RPS)], acc_sh.at[pl.ds(r0, RPS)])
    plsc.subcore_barrier()

    for b in range(NBUF):
        pltpu.async_copy(idx_hbm.at[cid, sid, b], idx_v.at[b], isem.at[b])
    for b in range(NBUF):
        pltpu.make_async_copy(
            idx_hbm.at[cid, sid, 0], idx_v.at[b], isem.at[b]).wait()
        pltpu.async_copy(g_hbm.at[idx_v.at[b, 0]], rows_v.at[b], gsem.at[b])

    def step(s, carry):
        base = s * NBUF
        for b in range(NBUF):
            pltpu.make_async_copy(
                g_hbm.at[idx_v.at[0, 0]], rows_v.at[b], gsem.at[b]).wait()
            pltpu.async_copy(
                rows_v.at[b], acc_sh.at[idx_v.at[b, 1]], ssem.at[b], add=True)
        for b in range(NBUF):
            pltpu.make_async_copy(
                zer_hbm.at[pl.ds(0, CE)], rows_v.at[b], ssem.at[b]).wait()

            @pl.when(s < STEPS - 1)
            def _():
                pltpu.async_copy(
                    idx_hbm.at[cid, sid, base + NBUF + b], idx_v.at[b],
                    isem.at[b])
        for b in range(NBUF):
            @pl.when(s < STEPS - 1)
            def _():
                pltpu.make_async_copy(
                    idx_hbm.at[cid, sid, 0], idx_v.at[b], isem.at[b]).wait()
                pltpu.async_copy(
                    g_hbm.at[idx_v.at[b, 0]], rows_v.at[b], gsem.at[b])
        return carry

    lax.fori_loop(0, STEPS, step, 0)
    plsc.subcore_barrier()
    pltpu.sync_copy(acc_sh.at[pl.ds(r0, RPS)], out_hbm.at[cid, pl.ds(r0, RPS)])


_prop = functools.partial(
    pl.kernel,
    out_type=jax.ShapeDtypeStruct((NSC, NACC, D), jnp.float32),
    mesh=_mesh,
    scratch_types=[
        pltpu.VMEM((NBUF, 2, CE), jnp.int32),
        pltpu.VMEM((NBUF, CE, D), jnp.float32),
        pltpu.VMEM_SHARED((NACC, D), jnp.float32),
        pltpu.SemaphoreType.DMA((NBUF,)),
        pltpu.SemaphoreType.DMA((NBUF,)),
        pltpu.SemaphoreType.DMA((NBUF,)),
    ],
)(_prop_body)


def _deg_body(dst_hbm, zer_hbm, ones_hbm, out_hbm, dst_v, ones_v, acc_sh, ssem):
    cid = lax.axis_index("c")
    sid = lax.axis_index("s")
    r0 = sid * RPS
    pltpu.sync_copy(zer_hbm.at[pl.ds(r0, RPS)], acc_sh.at[pl.ds(r0, RPS)])
    pltpu.sync_copy(ones_hbm, ones_v)
    pltpu.sync_copy(dst_hbm.at[cid, sid], dst_v)
    plsc.subcore_barrier()

    def step(s, carry):
        for b in range(8):
            j = s * 8 + b
            pltpu.async_copy(ones_v, acc_sh.at[dst_v.at[j]], ssem, add=True)
        for b in range(8):
            pltpu.make_async_copy(
                zer_hbm.at[pl.ds(0, CE)], ones_v, ssem).wait()
        return carry

    lax.fori_loop(0, CHN // 8, step, 0)
    plsc.subcore_barrier()
    pltpu.sync_copy(acc_sh.at[pl.ds(r0, RPS)], out_hbm.at[cid, pl.ds(r0, RPS)])


_deg = functools.partial(
    pl.kernel,
    out_type=jax.ShapeDtypeStruct((NSC, NACC, D), jnp.float32),
    mesh=_mesh,
    scratch_types=[
        pltpu.VMEM((CHN, CE), jnp.int32),
        pltpu.VMEM((CE, D), jnp.float32),
        pltpu.VMEM_SHARED((NACC, D), jnp.float32),
        pltpu.SemaphoreType.DMA,
    ],
)(_deg_body)


def _mm(a, b):
    return jnp.dot(a, b, preferred_element_type=jnp.float32)


def _dense0_body(x_ref, w0_ref, b0_ref, wg_ref, degp_ref,
                 g_ref, acc_ref, dinv_ref):
    deg = degp_ref[0, :, 0:1] + degp_ref[1, :, 0:1]
    dinv = jnp.where(deg > 0, lax.rsqrt(jnp.maximum(deg, 1.0)), 0.0)
    h = jnp.maximum(_mm(x_ref[...], w0_ref[...]) + b0_ref[...], 0.0)
    acc_ref[...] = _mm(h, wg_ref[...])
    g_ref[...] = h * dinv
    dinv_ref[...] = dinv


_dense0 = pl.pallas_call(
    _dense0_body,
    grid=(N // R,),
    in_specs=[
        pl.BlockSpec((R, D), lambda i: (i, 0)),
        pl.BlockSpec((D, D), lambda i: (0, 0)),
        pl.BlockSpec((1, D), lambda i: (0, 0)),
        pl.BlockSpec((D, D), lambda i: (0, 0)),
        pl.BlockSpec((NSC, R, D), lambda i: (0, i, 0)),
    ],
    out_specs=[
        pl.BlockSpec((R, D), lambda i: (i, 0)),
        pl.BlockSpec((R, D), lambda i: (i, 0)),
        pl.BlockSpec((R, 1), lambda i: (i, 0)),
    ],
    out_shape=[
        jax.ShapeDtypeStruct((N, D), jnp.float32),
        jax.ShapeDtypeStruct((N, D), jnp.float32),
        jax.ShapeDtypeStruct((N, 1), jnp.float32),
    ],
)


def _hop_mid_body(p_ref, dinv_ref, wk_ref, acc_ref, accout_ref, g_ref):
    dinv = dinv_ref[...]
    hk = (p_ref[0] + p_ref[1]) * dinv
    accout_ref[...] = acc_ref[...] + _mm(hk, wk_ref[...])
    g_ref[...] = hk * dinv


_hop_mid = pl.pallas_call(
    _hop_mid_body,
    grid=(N // R,),
    in_specs=[
        pl.BlockSpec((NSC, R, D), lambda i: (0, i, 0)),
        pl.BlockSpec((R, 1), lambda i: (i, 0)),
        pl.BlockSpec((D, D), lambda i: (0, 0)),
        pl.BlockSpec((R, D), lambda i: (i, 0)),
    ],
    out_specs=[
        pl.BlockSpec((R, D), lambda i: (i, 0)),
        pl.BlockSpec((R, D), lambda i: (i, 0)),
    ],
    out_shape=[
        jax.ShapeDtypeStruct((N, D), jnp.float32),
        jax.ShapeDtypeStruct((N, D), jnp.float32),
    ],
)


def _hop_last_body(p_ref, dinv_ref, wk_ref, acc_ref, bg_ref, wn_ref,
                   accout_ref, g_ref):
    dinv = dinv_ref[...]
    hk = (p_ref[0] + p_ref[1]) * dinv
    t = jnp.maximum(acc_ref[...] + _mm(hk, wk_ref[...]) + bg_ref[...], 0.0)
    accout_ref[...] = _mm(t, wn_ref[...])
    g_ref[...] = t * dinv


_hop_last = pl.pallas_call(
    _hop_last_body,
    grid=(N // R,),
    in_specs=[
        pl.BlockSpec((NSC, R, D), lambda i: (0, i, 0)),
        pl.BlockSpec((R, 1), lambda i: (i, 0)),
        pl.BlockSpec((D, D), lambda i: (0, 0)),
        pl.BlockSpec((R, D), lambda i: (i, 0)),
        pl.BlockSpec((1, D), lambda i: (0, 0)),
        pl.BlockSpec((D, D), lambda i: (0, 0)),
    ],
    out_specs=[
        pl.BlockSpec((R, D), lambda i: (i, 0)),
        pl.BlockSpec((R, D), lambda i: (i, 0)),
    ],
    out_shape=[
        jax.ShapeDtypeStruct((N, D), jnp.float32),
        jax.ShapeDtypeStruct((N, D), jnp.float32),
    ],
)


def _hop_final_body(p_ref, dinv_ref, wk_ref, acc_ref, bg_ref,
                    wm0_ref, bm0_ref, wm1_ref, bm1_ref, wo_ref, bo_ref,
                    out_ref):
    dinv = dinv_ref[...]
    hk = (p_ref[0] + p_ref[1]) * dinv
    t = jnp.maximum(acc_ref[...] + _mm(hk, wk_ref[...]) + bg_ref[...], 0.0)
    m = jnp.maximum(_mm(t, wm0_ref[...]) + bm0_ref[...], 0.0)
    m = jnp.maximum(_mm(m, wm1_ref[...]) + bm1_ref[...], 0.0)
    out_ref[...] = jnp.maximum(_mm(m, wo_ref[...]) + bo_ref[...], 0.0)


_hop_final = pl.pallas_call(
    _hop_final_body,
    grid=(N // R,),
    in_specs=[
        pl.BlockSpec((NSC, R, D), lambda i: (0, i, 0)),
        pl.BlockSpec((R, 1), lambda i: (i, 0)),
        pl.BlockSpec((D, D), lambda i: (0, 0)),
        pl.BlockSpec((R, D), lambda i: (i, 0)),
        pl.BlockSpec((1, D), lambda i: (0, 0)),
        pl.BlockSpec((D, D), lambda i: (0, 0)),
        pl.BlockSpec((1, D), lambda i: (0, 0)),
        pl.BlockSpec((D, D), lambda i: (0, 0)),
        pl.BlockSpec((1, D), lambda i: (0, 0)),
        pl.BlockSpec((D, 1), lambda i: (0, 0)),
        pl.BlockSpec((1, 1), lambda i: (0, 0)),
    ],
    out_specs=pl.BlockSpec((R, 1), lambda i: (i, 0)),
    out_shape=jax.ShapeDtypeStruct((N, 1), jnp.float32),
)


def kernel(x, edge_index, batch, W_lin0, b_lin0,
           Wg0_0, Wg0_1, Wg0_2, Wg0_3, bg0,
           Wg1_0, Wg1_1, Wg1_2, Wg1_3, bg1,
           W_mlp0, b_mlp0, W_mlp1, b_mlp1, W_out, b_out):
    src = edge_index[0]
    dst = edge_index[1]
    src_p = jnp.concatenate(
        [src, jnp.zeros((EPAD,), jnp.int32)]).reshape(NSC, NSUB, CHN, CE)
    dst_p = jnp.concatenate(
        [dst, jnp.full((EPAD,), N, jnp.int32)]).reshape(NSC, NSUB, CHN, CE)
    idx_p = jnp.stack([src_p, dst_p], axis=3)
    z128 = jnp.zeros((NACC, D), jnp.float32)
    ones128 = jnp.ones((CE, D), jnp.float32)

    degp = _deg(dst_p, z128, ones128)
    g, acc, dinv = _dense0(x, W_lin0, b_lin0.reshape(1, D), Wg0_0, degp)

    for wk in (Wg0_1, Wg0_2):
        p = _prop(g, idx_p, z128)
        acc, g = _hop_mid(p, dinv, wk, acc)
    p = _prop(g, idx_p, z128)
    acc, g = _hop_last(p, dinv, Wg0_3, acc, bg0.reshape(1, D), Wg1_0)

    for wk in (Wg1_1, Wg1_2):
        p = _prop(g, idx_p, z128)
        acc, g = _hop_mid(p, dinv, wk, acc)
    p = _prop(g, idx_p, z128)
    out = _hop_final(p, dinv, Wg1_3, acc, bg1.reshape(1, D),
                     W_mlp0, b_mlp0.reshape(1, D),
                     W_mlp1, b_mlp1.reshape(1, D),
                     W_out, b_out.reshape(1, 1))
    return out


# speedup vs baseline: 1.4900x; 1.4900x over previous
import functools

import jax
import jax.numpy as jnp
from jax import lax
from jax.experimental import pallas as pl
from jax.experimental.pallas import tpu as pltpu
from jax.experimental.pallas import tpu_sc as plsc

N = 10000
D = 128
E = 320000
NACC = 10112
NSC = 2
NSUB = 16
NW = NSC * NSUB
CH = 79
EPW = CH * 128
EPAD = NW * EPW - E
RPS = NACC // NSUB
R = 1000

_mesh = plsc.VectorSubcoreMesh(core_axis_name="c", subcore_axis_name="s")


def _prop_body(g_hbm, src_hbm, dst_hbm, zer_hbm, out_hbm,
               src_v, dst_v, rows_v, acc_sh, sem):
    cid = lax.axis_index("c")
    sid = lax.axis_index("s")
    r0 = sid * RPS
    pltpu.sync_copy(zer_hbm.at[pl.ds(r0, RPS)], acc_sh.at[pl.ds(r0, RPS)])
    pltpu.sync_copy(src_hbm.at[cid, sid], src_v)
    pltpu.sync_copy(dst_hbm.at[cid, sid], dst_v)
    plsc.subcore_barrier()

    def body(j, carry):
        pltpu.async_copy(g_hbm.at[src_v.at[j]], rows_v, sem).wait()
        pltpu.sync_copy(rows_v, acc_sh.at[dst_v.at[j]], add=True)
        return carry

    lax.fori_loop(0, CH, body, 0)
    plsc.subcore_barrier()
    pltpu.sync_copy(acc_sh.at[pl.ds(r0, RPS)], out_hbm.at[cid, pl.ds(r0, RPS)])


_prop = functools.partial(
    pl.kernel,
    out_type=jax.ShapeDtypeStruct((NSC, NACC, D), jnp.float32),
    mesh=_mesh,
    scratch_types=[
        pltpu.VMEM((CH, 128), jnp.int32),
        pltpu.VMEM((CH, 128), jnp.int32),
        pltpu.VMEM((128, D), jnp.float32),
        pltpu.VMEM_SHARED((NACC, D), jnp.float32),
        pltpu.SemaphoreType.DMA,
    ],
)(_prop_body)


def _deg_body(dst_hbm, zer_hbm, ones_hbm, out_hbm, dst_v, ones_v, acc_sh):
    cid = lax.axis_index("c")
    sid = lax.axis_index("s")
    r0 = sid * RPS
    pltpu.sync_copy(zer_hbm.at[pl.ds(r0, RPS)], acc_sh.at[pl.ds(r0, RPS)])
    pltpu.sync_copy(ones_hbm, ones_v)
    pltpu.sync_copy(dst_hbm.at[cid, sid], dst_v)
    plsc.subcore_barrier()

    def body(j, carry):
        pltpu.sync_copy(ones_v, acc_sh.at[dst_v.at[j]], add=True)
        return carry

    lax.fori_loop(0, CH, body, 0)
    plsc.subcore_barrier()
    pltpu.sync_copy(acc_sh.at[pl.ds(r0, RPS)], out_hbm.at[cid, pl.ds(r0, RPS)])


_deg = functools.partial(
    pl.kernel,
    out_type=jax.ShapeDtypeStruct((NSC, NACC, 64), jnp.float32),
    mesh=_mesh,
    scratch_types=[
        pltpu.VMEM((CH, 128), jnp.int32),
        pltpu.VMEM((128, 64), jnp.float32),
        pltpu.VMEM_SHARED((NACC, 64), jnp.float32),
    ],
)(_deg_body)


def _mm(a, b):
    return jnp.dot(a, b, preferred_element_type=jnp.float32)


def _dense_h_body(x_ref, w0_ref, b0_ref, wg_ref, h_ref, acc_ref):
    h = jnp.maximum(_mm(x_ref[...], w0_ref[...]) + b0_ref[...], 0.0)
    h_ref[...] = h
    acc_ref[...] = _mm(h, wg_ref[...])


_dense_h = pl.pallas_call(
    _dense_h_body,
    grid=(N // R,),
    in_specs=[
        pl.BlockSpec((R, D), lambda i: (i, 0)),
        pl.BlockSpec((D, D), lambda i: (0, 0)),
        pl.BlockSpec((1, D), lambda i: (0, 0)),
        pl.BlockSpec((D, D), lambda i: (0, 0)),
    ],
    out_specs=[
        pl.BlockSpec((R, D), lambda i: (i, 0)),
        pl.BlockSpec((R, D), lambda i: (i, 0)),
    ],
    out_shape=[
        jax.ShapeDtypeStruct((N, D), jnp.float32),
        jax.ShapeDtypeStruct((N, D), jnp.float32),
    ],
)


def _dense_scale_body(h_ref, degp_ref, g_ref, dinv_ref):
    deg = degp_ref[0, :, 0:1] + degp_ref[1, :, 0:1]
    dinv = jnp.where(deg > 0, lax.rsqrt(jnp.maximum(deg, 1.0)), 0.0)
    g_ref[...] = h_ref[...] * dinv
    dinv_ref[...] = dinv


_dense_scale = pl.pallas_call(
    _dense_scale_body,
    grid=(N // R,),
    in_specs=[
        pl.BlockSpec((R, D), lambda i: (i, 0)),
        pl.BlockSpec((NSC, R, 64), lambda i: (0, i, 0)),
    ],
    out_specs=[
        pl.BlockSpec((R, D), lambda i: (i, 0)),
        pl.BlockSpec((R, 1), lambda i: (i, 0)),
    ],
    out_shape=[
        jax.ShapeDtypeStruct((N, D), jnp.float32),
        jax.ShapeDtypeStruct((N, 1), jnp.float32),
    ],
)


def _hop_g_body(p_ref, dinv_ref, g_ref):
    dinv = dinv_ref[...]
    g_ref[...] = (p_ref[0] + p_ref[1]) * (dinv * dinv)


_hop_g = pl.pallas_call(
    _hop_g_body,
    grid=(N // R,),
    in_specs=[
        pl.BlockSpec((NSC, R, D), lambda i: (0, i, 0)),
        pl.BlockSpec((R, 1), lambda i: (i, 0)),
    ],
    out_specs=pl.BlockSpec((R, D), lambda i: (i, 0)),
    out_shape=jax.ShapeDtypeStruct((N, D), jnp.float32),
)


def _hop_mm_body(p_ref, dinv_ref, wk_ref, acc_ref, accout_ref):
    hk = (p_ref[0] + p_ref[1]) * dinv_ref[...]
    accout_ref[...] = acc_ref[...] + _mm(hk, wk_ref[...])


_hop_mm = pl.pallas_call(
    _hop_mm_body,
    grid=(N // R,),
    in_specs=[
        pl.BlockSpec((NSC, R, D), lambda i: (0, i, 0)),
        pl.BlockSpec((R, 1), lambda i: (i, 0)),
        pl.BlockSpec((D, D), lambda i: (0, 0)),
        pl.BlockSpec((R, D), lambda i: (i, 0)),
    ],
    out_specs=pl.BlockSpec((R, D), lambda i: (i, 0)),
    out_shape=jax.ShapeDtypeStruct((N, D), jnp.float32),
)


def _hop_last_body(p_ref, dinv_ref, wk_ref, acc_ref, bg_ref, wn_ref,
                   accout_ref, g_ref):
    dinv = dinv_ref[...]
    hk = (p_ref[0] + p_ref[1]) * dinv
    t = jnp.maximum(acc_ref[...] + _mm(hk, wk_ref[...]) + bg_ref[...], 0.0)
    accout_ref[...] = _mm(t, wn_ref[...])
    g_ref[...] = t * dinv


_hop_last = pl.pallas_call(
    _hop_last_body,
    grid=(N // R,),
    in_specs=[
        pl.BlockSpec((NSC, R, D), lambda i: (0, i, 0)),
        pl.BlockSpec((R, 1), lambda i: (i, 0)),
        pl.BlockSpec((D, D), lambda i: (0, 0)),
        pl.BlockSpec((R, D), lambda i: (i, 0)),
        pl.BlockSpec((1, D), lambda i: (0, 0)),
        pl.BlockSpec((D, D), lambda i: (0, 0)),
    ],
    out_specs=[
        pl.BlockSpec((R, D), lambda i: (i, 0)),
        pl.BlockSpec((R, D), lambda i: (i, 0)),
    ],
    out_shape=[
        jax.ShapeDtypeStruct((N, D), jnp.float32),
        jax.ShapeDtypeStruct((N, D), jnp.float32),
    ],
)


def _hop_final_body(p_ref, dinv_ref, wk_ref, acc_ref, bg_ref,
                    wm0_ref, bm0_ref, wm1_ref, bm1_ref, wo_ref, bo_ref,
                    out_ref):
    dinv = dinv_ref[...]
    hk = (p_ref[0] + p_ref[1]) * dinv
    t = jnp.maximum(acc_ref[...] + _mm(hk, wk_ref[...]) + bg_ref[...], 0.0)
    m = jnp.maximum(_mm(t, wm0_ref[...]) + bm0_ref[...], 0.0)
    m = jnp.maximum(_mm(m, wm1_ref[...]) + bm1_ref[...], 0.0)
    out_ref[...] = jnp.maximum(_mm(m, wo_ref[...]) + bo_ref[...], 0.0)


_hop_final = pl.pallas_call(
    _hop_final_body,
    grid=(N // R,),
    in_specs=[
        pl.BlockSpec((NSC, R, D), lambda i: (0, i, 0)),
        pl.BlockSpec((R, 1), lambda i: (i, 0)),
        pl.BlockSpec((D, D), lambda i: (0, 0)),
        pl.BlockSpec((R, D), lambda i: (i, 0)),
        pl.BlockSpec((1, D), lambda i: (0, 0)),
        pl.BlockSpec((D, D), lambda i: (0, 0)),
        pl.BlockSpec((1, D), lambda i: (0, 0)),
        pl.BlockSpec((D, D), lambda i: (0, 0)),
        pl.BlockSpec((1, D), lambda i: (0, 0)),
        pl.BlockSpec((D, 1), lambda i: (0, 0)),
        pl.BlockSpec((1, 1), lambda i: (0, 0)),
    ],
    out_specs=pl.BlockSpec((R, 1), lambda i: (i, 0)),
    out_shape=jax.ShapeDtypeStruct((N, 1), jnp.float32),
)


def kernel(x, edge_index, batch, W_lin0, b_lin0,
           Wg0_0, Wg0_1, Wg0_2, Wg0_3, bg0,
           Wg1_0, Wg1_1, Wg1_2, Wg1_3, bg1,
           W_mlp0, b_mlp0, W_mlp1, b_mlp1, W_out, b_out):
    src = edge_index[0]
    dst = edge_index[1]
    src_p = jnp.concatenate(
        [src, jnp.zeros((EPAD,), jnp.int32)]).reshape(NSC, NSUB, CH, 128)
    dst_p = jnp.concatenate(
        [dst, jnp.full((EPAD,), N, jnp.int32)]).reshape(NSC, NSUB, CH, 128)
    z128 = jnp.zeros((NACC, D), jnp.float32)
    z64 = jnp.zeros((NACC, 64), jnp.float32)
    ones64 = jnp.ones((128, 64), jnp.float32)

    degp = _deg(dst_p, z64, ones64)
    h, acc = _dense_h(x, W_lin0, b_lin0.reshape(1, D), Wg0_0)
    g, dinv = _dense_scale(h, degp)

    for wk in (Wg0_1, Wg0_2):
        p = _prop(g, src_p, dst_p, z128)
        g = _hop_g(p, dinv)
        acc = _hop_mm(p, dinv, wk, acc)
    p = _prop(g, src_p, dst_p, z128)
    acc, g = _hop_last(p, dinv, Wg0_3, acc, bg0.reshape(1, D), Wg1_0)

    for wk in (Wg1_1, Wg1_2):
        p = _prop(g, src_p, dst_p, z128)
        g = _hop_g(p, dinv)
        acc = _hop_mm(p, dinv, wk, acc)
    p = _prop(g, src_p, dst_p, z128)
    out = _hop_final(p, dinv, Wg1_3, acc, bg1.reshape(1, D),
                     W_mlp0, b_mlp0.reshape(1, D),
                     W_mlp1, b_mlp1.reshape(1, D),
                     W_out, b_out.reshape(1, 1))
    return out


# speedup vs baseline: 1.5189x; 1.0194x over previous
import functools

import jax
import jax.numpy as jnp
from jax import lax
from jax.experimental import pallas as pl
from jax.experimental.pallas import tpu as pltpu
from jax.experimental.pallas import tpu_sc as plsc

N = 10000
D = 128
E = 320000
NACC = 10112
NSC = 2
NSUB = 16
NW = NSC * NSUB
CH = 79
EPW = CH * 128
EPAD = NW * EPW - E
RPS = NACC // NSUB
R = 1000

_mesh = plsc.VectorSubcoreMesh(core_axis_name="c", subcore_axis_name="s")


def _prop_body(g_hbm, src_hbm, dst_hbm, zer_hbm, out_hbm,
               src_v, dst_v, rows_v, acc_sh, sem):
    cid = lax.axis_index("c")
    sid = lax.axis_index("s")
    r0 = sid * RPS
    pltpu.sync_copy(zer_hbm.at[pl.ds(r0, RPS)], acc_sh.at[pl.ds(r0, RPS)])
    pltpu.sync_copy(src_hbm.at[cid, sid], src_v)
    pltpu.sync_copy(dst_hbm.at[cid, sid], dst_v)
    plsc.subcore_barrier()

    def body(j, carry):
        pltpu.async_copy(g_hbm.at[src_v.at[j]], rows_v, sem).wait()
        pltpu.sync_copy(rows_v, acc_sh.at[dst_v.at[j]], add=True)
        return carry

    lax.fori_loop(0, CH, body, 0)
    plsc.subcore_barrier()
    pltpu.sync_copy(acc_sh.at[pl.ds(r0, RPS)], out_hbm.at[cid, pl.ds(r0, RPS)])


_prop = functools.partial(
    pl.kernel,
    out_type=jax.ShapeDtypeStruct((NSC, NACC, D), jnp.float32),
    mesh=_mesh,
    scratch_types=[
        pltpu.VMEM((CH, 128), jnp.int32),
        pltpu.VMEM((CH, 128), jnp.int32),
        pltpu.VMEM((128, D), jnp.float32),
        pltpu.VMEM_SHARED((NACC, D), jnp.float32),
        pltpu.SemaphoreType.DMA,
    ],
)(_prop_body)


def _deg_body(dst_hbm, zer_hbm, ones_hbm, out_hbm, dst_v, ones_v, acc_sh):
    cid = lax.axis_index("c")
    sid = lax.axis_index("s")
    r0 = sid * RPS
    pltpu.sync_copy(zer_hbm.at[pl.ds(r0, RPS)], acc_sh.at[pl.ds(r0, RPS)])
    pltpu.sync_copy(ones_hbm, ones_v)
    pltpu.sync_copy(dst_hbm.at[cid, sid], dst_v)
    plsc.subcore_barrier()

    def body(j, carry):
        pltpu.sync_copy(ones_v, acc_sh.at[dst_v.at[j]], add=True)
        return carry

    lax.fori_loop(0, CH, body, 0)
    plsc.subcore_barrier()
    pltpu.sync_copy(acc_sh.at[pl.ds(r0, RPS)], out_hbm.at[cid, pl.ds(r0, RPS)])


_deg = functools.partial(
    pl.kernel,
    out_type=jax.ShapeDtypeStruct((NSC, NACC, D), jnp.float32),
    mesh=_mesh,
    scratch_types=[
        pltpu.VMEM((CH, 128), jnp.int32),
        pltpu.VMEM((128, D), jnp.float32),
        pltpu.VMEM_SHARED((NACC, D), jnp.float32),
    ],
)(_deg_body)


def _mm(a, b):
    return jnp.dot(a, b, preferred_element_type=jnp.float32)


def _dense_h_body(x_ref, w0_ref, b0_ref, wg_ref, h_ref, acc_ref):
    h = jnp.maximum(_mm(x_ref[...], w0_ref[...]) + b0_ref[...], 0.0)
    h_ref[...] = h
    acc_ref[...] = _mm(h, wg_ref[...])


_dense_h = pl.pallas_call(
    _dense_h_body,
    grid=(N // R,),
    in_specs=[
        pl.BlockSpec((R, D), lambda i: (i, 0)),
        pl.BlockSpec((D, D), lambda i: (0, 0)),
        pl.BlockSpec((1, D), lambda i: (0, 0)),
        pl.BlockSpec((D, D), lambda i: (0, 0)),
    ],
    out_specs=[
        pl.BlockSpec((R, D), lambda i: (i, 0)),
        pl.BlockSpec((R, D), lambda i: (i, 0)),
    ],
    out_shape=[
        jax.ShapeDtypeStruct((N, D), jnp.float32),
        jax.ShapeDtypeStruct((N, D), jnp.float32),
    ],
)


def _dense_scale_body(h_ref, degp_ref, g_ref, dinv_ref):
    deg = degp_ref[0, :, 0:1] + degp_ref[1, :, 0:1]
    dinv = jnp.where(deg > 0, lax.rsqrt(jnp.maximum(deg, 1.0)), 0.0)
    g_ref[...] = h_ref[...] * dinv
    dinv_ref[...] = dinv


_dense_scale = pl.pallas_call(
    _dense_scale_body,
    grid=(N // R,),
    in_specs=[
        pl.BlockSpec((R, D), lambda i: (i, 0)),
        pl.BlockSpec((NSC, R, D), lambda i: (0, i, 0)),
    ],
    out_specs=[
        pl.BlockSpec((R, D), lambda i: (i, 0)),
        pl.BlockSpec((R, 1), lambda i: (i, 0)),
    ],
    out_shape=[
        jax.ShapeDtypeStruct((N, D), jnp.float32),
        jax.ShapeDtypeStruct((N, 1), jnp.float32),
    ],
)


def _hop_g_body(p_ref, dinv_ref, g_ref):
    dinv = dinv_ref[...]
    g_ref[...] = (p_ref[0] + p_ref[1]) * (dinv * dinv)


_hop_g = pl.pallas_call(
    _hop_g_body,
    grid=(N // R,),
    in_specs=[
        pl.BlockSpec((NSC, R, D), lambda i: (0, i, 0)),
        pl.BlockSpec((R, 1), lambda i: (i, 0)),
    ],
    out_specs=pl.BlockSpec((R, D), lambda i: (i, 0)),
    out_shape=jax.ShapeDtypeStruct((N, D), jnp.float32),
)


def _hop_mm_body(p_ref, dinv_ref, wk_ref, acc_ref, accout_ref):
    hk = (p_ref[0] + p_ref[1]) * dinv_ref[...]
    accout_ref[...] = acc_ref[...] + _mm(hk, wk_ref[...])


_hop_mm = pl.pallas_call(
    _hop_mm_body,
    grid=(N // R,),
    in_specs=[
        pl.BlockSpec((NSC, R, D), lambda i: (0, i, 0)),
        pl.BlockSpec((R, 1), lambda i: (i, 0)),
        pl.BlockSpec((D, D), lambda i: (0, 0)),
        pl.BlockSpec((R, D), lambda i: (i, 0)),
    ],
    out_specs=pl.BlockSpec((R, D), lambda i: (i, 0)),
    out_shape=jax.ShapeDtypeStruct((N, D), jnp.float32),
)


def _hop_last_body(p_ref, dinv_ref, wk_ref, acc_ref, bg_ref, wn_ref,
                   accout_ref, g_ref):
    dinv = dinv_ref[...]
    hk = (p_ref[0] + p_ref[1]) * dinv
    t = jnp.maximum(acc_ref[...] + _mm(hk, wk_ref[...]) + bg_ref[...], 0.0)
    accout_ref[...] = _mm(t, wn_ref[...])
    g_ref[...] = t * dinv


_hop_last = pl.pallas_call(
    _hop_last_body,
    grid=(N // R,),
    in_specs=[
        pl.BlockSpec((NSC, R, D), lambda i: (0, i, 0)),
        pl.BlockSpec((R, 1), lambda i: (i, 0)),
        pl.BlockSpec((D, D), lambda i: (0, 0)),
        pl.BlockSpec((R, D), lambda i: (i, 0)),
        pl.BlockSpec((1, D), lambda i: (0, 0)),
        pl.BlockSpec((D, D), lambda i: (0, 0)),
    ],
    out_specs=[
        pl.BlockSpec((R, D), lambda i: (i, 0)),
        pl.BlockSpec((R, D), lambda i: (i, 0)),
    ],
    out_shape=[
        jax.ShapeDtypeStruct((N, D), jnp.float32),
        jax.ShapeDtypeStruct((N, D), jnp.float32),
    ],
)


def _hop_final_body(p_ref, dinv_ref, wk_ref, acc_ref, bg_ref,
                    wm0_ref, bm0_ref, wm1_ref, bm1_ref, wo_ref, bo_ref,
                    out_ref):
    dinv = dinv_ref[...]
    hk = (p_ref[0] + p_ref[1]) * dinv
    t = jnp.maximum(acc_ref[...] + _mm(hk, wk_ref[...]) + bg_ref[...], 0.0)
    m = jnp.maximum(_mm(t, wm0_ref[...]) + bm0_ref[...], 0.0)
    m = jnp.maximum(_mm(m, wm1_ref[...]) + bm1_ref[...], 0.0)
    out_ref[...] = jnp.maximum(_mm(m, wo_ref[...]) + bo_ref[...], 0.0)


_hop_final = pl.pallas_call(
    _hop_final_body,
    grid=(N // R,),
    in_specs=[
        pl.BlockSpec((NSC, R, D), lambda i: (0, i, 0)),
        pl.BlockSpec((R, 1), lambda i: (i, 0)),
        pl.BlockSpec((D, D), lambda i: (0, 0)),
        pl.BlockSpec((R, D), lambda i: (i, 0)),
        pl.BlockSpec((1, D), lambda i: (0, 0)),
        pl.BlockSpec((D, D), lambda i: (0, 0)),
        pl.BlockSpec((1, D), lambda i: (0, 0)),
        pl.BlockSpec((D, D), lambda i: (0, 0)),
        pl.BlockSpec((1, D), lambda i: (0, 0)),
        pl.BlockSpec((D, 1), lambda i: (0, 0)),
        pl.BlockSpec((1, 1), lambda i: (0, 0)),
    ],
    out_specs=pl.BlockSpec((R, 1), lambda i: (i, 0)),
    out_shape=jax.ShapeDtypeStruct((N, 1), jnp.float32),
)


def kernel(x, edge_index, batch, W_lin0, b_lin0,
           Wg0_0, Wg0_1, Wg0_2, Wg0_3, bg0,
           Wg1_0, Wg1_1, Wg1_2, Wg1_3, bg1,
           W_mlp0, b_mlp0, W_mlp1, b_mlp1, W_out, b_out):
    src = edge_index[0]
    dst = edge_index[1]
    src_p = jnp.concatenate(
        [src, jnp.zeros((EPAD,), jnp.int32)]).reshape(NSC, NSUB, CH, 128)
    dst_p = jnp.concatenate(
        [dst, jnp.full((EPAD,), N, jnp.int32)]).reshape(NSC, NSUB, CH, 128)
    z128 = jnp.zeros((NACC, D), jnp.float32)
    ones128 = jnp.ones((128, D), jnp.float32)

    degp = _deg(dst_p, z128, ones128)
    h, acc = _dense_h(x, W_lin0, b_lin0.reshape(1, D), Wg0_0)
    g, dinv = _dense_scale(h, degp)

    for wk in (Wg0_1, Wg0_2):
        p = _prop(g, src_p, dst_p, z128)
        g = _hop_g(p, dinv)
        acc = _hop_mm(p, dinv, wk, acc)
    p = _prop(g, src_p, dst_p, z128)
    acc, g = _hop_last(p, dinv, Wg0_3, acc, bg0.reshape(1, D), Wg1_0)

    for wk in (Wg1_1, Wg1_2):
        p = _prop(g, src_p, dst_p, z128)
        g = _hop_g(p, dinv)
        acc = _hop_mm(p, dinv, wk, acc)
    p = _prop(g, src_p, dst_p, z128)
    out = _hop_final(p, dinv, Wg1_3, acc, bg1.reshape(1, D),
                     W_mlp0, b_mlp0.reshape(1, D),
                     W_mlp1, b_mlp1.reshape(1, D),
                     W_out, b_out.reshape(1, 1))
    return out
